# baseline (speedup 1.0000x reference)
import functools

import jax
import jax.numpy as jnp
from jax import lax
from jax.experimental import pallas as pl
from jax.experimental.pallas import tpu as pltpu
from jax.experimental.pallas import tpu_sc as plsc

N_OUT = 4096
N_IN = 4096
D = 4
K = 256
NUM_IDX = N_OUT * N_IN // D

NC = 2
NS = 16
CHUNK = 4096
SC_CHUNK = NS * CHUNK
NCHUNK = NUM_IDX // NC // SC_CHUNK
OUT_W = CHUNK * D
SC_OUT_W = SC_CHUNK * D


def _body(cb_hbm, idx_hbm, out_hbm,
          cb_v, idx_v0, idx_v1, out_v0, out_v1, shared0, shared1,
          cb_sem, in_sem0, in_sem1, xb_sem0, xb_sem1, big_sem0, big_sem1):
    c = lax.axis_index("c")
    s = lax.axis_index("s")
    sc_base = c * (NUM_IDX // NC)

    pltpu.async_copy(cb_hbm, cb_v, cb_sem).wait()

    idx_bufs = (idx_v0, idx_v1)
    out_bufs = (out_v0, out_v1)
    shareds = (shared0, shared1)
    in_sems = (in_sem0, in_sem1)
    xb_sems = (xb_sem0, xb_sem1)
    big_sems = (big_sem0, big_sem1)

    lane = lax.iota(jnp.int32, 16)
    st_base = lane * 4

    def start_in(g):
        b = g % 2
        return pltpu.async_copy(
            idx_hbm.at[pl.ds(sc_base + g * SC_CHUNK + s * CHUNK, CHUNK)],
            idx_bufs[b], in_sems[b])

    def start_xb(g):
        b = g % 2
        return pltpu.async_copy(
            out_bufs[b], shareds[b].at[pl.ds(s * OUT_W, OUT_W)], xb_sems[b])

    def big_desc(g):
        b = g % 2
        return pltpu.make_async_copy(
            shareds[b],
            out_hbm.at[pl.ds((sc_base + g * SC_CHUNK) * D, SC_OUT_W)],
            big_sems[b])

    def compute(idx_ref, out_ref):
        @plsc.parallel_loop(0, CHUNK // 16, unroll=1)
        def body(i):
            a = idx_ref[pl.ds(i * 16, 16)]
            w = a * 4
            ob = i * 64
            vals = [plsc.load_gather(cb_v, [w + j]) for j in range(D)]
            for j in range(D):
                plsc.store_scatter(out_ref, [st_base + (ob + j)], vals[j])

    in_copies = [None, None]
    xb_copies = [None, None]
    in_copies[0] = start_in(0)
    for g in range(NCHUNK):
        b = g % 2
        if g + 1 < NCHUNK:
            in_copies[1 - b] = start_in(g + 1)
        in_copies[b].wait()
        compute(idx_bufs[b], out_bufs[b])
        if g >= 1:
            xb_copies[1 - b].wait()
            plsc.subcore_barrier()

            @pl.when(s == 0)
            def _():
                big_desc(g - 1).start()
        if g >= 2:
            @pl.when(s == 0)
            def _():
                big_desc(g - 2).wait()
            plsc.subcore_barrier()
        xb_copies[b] = start_xb(g)
    b = (NCHUNK - 1) % 2
    xb_copies[b].wait()
    plsc.subcore_barrier()

    @pl.when(s == 0)
    def _():
        big_desc(NCHUNK - 1).start()
        big_desc(NCHUNK - 2).wait()
        big_desc(NCHUNK - 1).wait()


_gather = functools.partial(
    pl.kernel,
    out_type=jax.ShapeDtypeStruct((NUM_IDX * D,), jnp.float32),
    mesh=plsc.VectorSubcoreMesh(core_axis_name="c", subcore_axis_name="s"),
    compiler_params=pltpu.CompilerParams(needs_layout_passes=False),
    scratch_types=[
        pltpu.VMEM((K * D,), jnp.float32),
        pltpu.VMEM((CHUNK,), jnp.int32),
        pltpu.VMEM((CHUNK,), jnp.int32),
        pltpu.VMEM((OUT_W,), jnp.float32),
        pltpu.VMEM((OUT_W,), jnp.float32),
        pltpu.VMEM_SHARED((SC_OUT_W,), jnp.float32),
        pltpu.VMEM_SHARED((SC_OUT_W,), jnp.float32),
        pltpu.SemaphoreType.DMA,
        pltpu.SemaphoreType.DMA,
        pltpu.SemaphoreType.DMA,
        pltpu.SemaphoreType.DMA,
        pltpu.SemaphoreType.DMA,
        pltpu.SemaphoreType.DMA,
        pltpu.SemaphoreType.DMA,
    ],
)(_body)


def kernel(centriods, assignments):
    out_flat = _gather(centriods.reshape(K * D), assignments)
    return out_flat.reshape(N_OUT, N_IN)
